# baseline (speedup 1.0000x reference)
import jax
import jax.numpy as jnp
from jax import lax
from jax.experimental import pallas as pl
from jax.experimental.pallas import tpu as pltpu
from jax.experimental.pallas import tpu_sc as plsc

NC = 2
NS = 16
NW = NC * NS
NB = 4
NBUF = 2


def _body(xr, table, out, idx0, idx1, rows0, rows1, gs0, gs1, ws0, ws1):
    idxs = (idx0, idx1)
    rows = (rows0, rows1)
    gsem = (gs0, gs1)
    wsem = (ws0, ws1)
    B, H, D = out.shape
    CHUNK = NB * H
    c = lax.axis_index("c")
    s = lax.axis_index("s")
    wid = s * NC + c
    chunks = B // (NW * NB)
    b_lo = wid * chunks * NB

    def load_dec(brow, b):
        pltpu.sync_copy(xr.at[pl.ds(brow * H, CHUNK)], idxs[b])
        for j in range(CHUNK // 16):
            sl = pl.ds(j * 16, 16)
            idxs[b][sl] = idxs[b][sl] - 1

    def fire_gathers(b):
        for i in range(NB):
            pltpu.async_copy(
                table.at[idxs[b].at[pl.ds(i * H, H)]], rows[b].at[i], gsem[b]
            )

    def drain_gathers(brow, b):
        pltpu.make_async_copy(out.at[pl.ds(brow, NB)], rows[b], gsem[b]).wait()

    def fire_wb(brow, b):
        pltpu.async_copy(rows[b], out.at[pl.ds(brow, NB)], wsem[b])

    def wait_wb(brow, b):
        pltpu.make_async_copy(rows[b], out.at[pl.ds(brow, NB)], wsem[b]).wait()

    for b in range(NBUF):
        load_dec(b_lo + b * NB, b)
        fire_gathers(b)

    @pl.loop(0, chunks - NBUF, step=NBUF)
    def main(G):
        for b in range(NBUF):
            brow = b_lo + (G + b) * NB
            drain_gathers(brow, b)
            fire_wb(brow, b)
            load_dec(brow + NBUF * NB, b)
            wait_wb(brow, b)
            fire_gathers(b)

    for b in range(NBUF):
        brow = b_lo + (chunks - NBUF + b) * NB
        drain_gathers(brow, b)
        fire_wb(brow, b)
        wait_wb(brow, b)


def kernel(x, table):
    B, H = x.shape
    V, D = table.shape
    xr = x.reshape(B * H)
    mesh = plsc.VectorSubcoreMesh(core_axis_name="c", subcore_axis_name="s")
    run = pl.kernel(
        _body,
        out_type=jax.ShapeDtypeStruct((B, H, D), jnp.float32),
        mesh=mesh,
        scratch_types=[
            pltpu.VMEM((NB * H,), jnp.int32),
            pltpu.VMEM((NB * H,), jnp.int32),
            pltpu.VMEM((NB, H, D), jnp.float32),
            pltpu.VMEM((NB, H, D), jnp.float32),
            pltpu.SemaphoreType.DMA,
            pltpu.SemaphoreType.DMA,
            pltpu.SemaphoreType.DMA,
            pltpu.SemaphoreType.DMA,
        ],
        compiler_params=pltpu.CompilerParams(use_tc_tiling_on_sc=False),
    )
    return run(xr, table)
